# baseline (speedup 1.0000x reference)
import functools

import jax
import jax.numpy as jnp
from jax import lax
from jax.experimental import pallas as pl
from jax.experimental.pallas import tpu as pltpu
from jax.experimental.pallas import tpu_sc as plsc

ENVOC2 = 500000
B = 4096
C = 30
CP = 32
D = 64
NT = 113

NC = 2
NS = 16
NW = NC * NS
BPW = B // NW
CHUNK = 16
NCHUNK = BPW // CHUNK
IDXC = CHUNK * C
GSL = 120
NG = IDXC // GSL
SKEW = 17


def _sc_body(table_hbm, widx_hbm, ctx_hbm, out_hbm,
             idx_v, pidx_v, rows_v, ctx_v, psa_v, psb_v, sco_v, sem):
    wid = lax.axis_index("s") * NC + lax.axis_index("c")
    zero16 = jnp.zeros((16,), jnp.float32)
    for pscr in (psa_v, psb_v):
        pscr[pl.ds(C * SKEW, 16)] = zero16
        pscr[pl.ds((C + 1) * SKEW, 16)] = zero16
    iota16 = lax.iota(jnp.int32, 16)
    tr_base = iota16 * SKEW

    for k in range(NCHUNK):
        bbase = wid * BPW + k * CHUNK
        ioff = pl.multiple_of(bbase * C, 8)
        pltpu.sync_copy(widx_hbm.at[pl.ds(ioff, IDXC)], idx_v.at[pl.ds(0, IDXC)])
        pltpu.sync_copy(ctx_hbm.at[pl.ds(bbase, CHUNK)], ctx_v)
        for t in range(IDXC // 16):
            pidx_v[pl.ds(t * 16, 16)] = lax.shift_right_logical(
                idx_v[pl.ds(t * 16, 16)], 1)
        cps = [
            pltpu.async_copy(
                table_hbm.at[pidx_v.at[pl.ds(j * GSL, GSL)]],
                rows_v.at[pl.ds(j * GSL, GSL)], sem)
            for j in range(NG)
        ]
        for cp in cps:
            cp.wait()

        def body_b(b, _):
            ctx = [ctx_v[b, pl.ds(i * 16, 16)] for i in range(4)]
            r0 = b * C
            for c in range(C):
                a0 = rows_v[r0 + c, pl.ds(0, 16)] * ctx[0]
                a1 = rows_v[r0 + c, pl.ds(64, 16)] * ctx[0]
                for i in range(1, 4):
                    a0 = a0 + rows_v[r0 + c, pl.ds(i * 16, 16)] * ctx[i]
                    a1 = a1 + rows_v[r0 + c, pl.ds(64 + i * 16, 16)] * ctx[i]
                psa_v[pl.ds(c * SKEW, 16)] = a0
                psb_v[pl.ds(c * SKEW, 16)] = a1
            for g in range(2):
                s0 = jnp.zeros((16,), jnp.float32)
                s1 = jnp.zeros((16,), jnp.float32)
                for l in range(16):
                    tr = tr_base + (g * 16 * SKEW + l)
                    s0 = s0 + plsc.load_gather(psa_v, [tr])
                    s1 = s1 + plsc.load_gather(psb_v, [tr])
                par = lax.rem(idx_v[pl.ds(r0 + g * 16, 16)], 2)
                sco_v[b, pl.ds(g * 16, 16)] = jnp.where(par == 1, s1, s0)
            return 0

        lax.fori_loop(0, CHUNK, body_b, 0)
        pltpu.sync_copy(sco_v, out_hbm.at[pl.ds(bbase, CHUNK)])


def _sc_scores(table2, widx_flat, ctx):
    mesh = plsc.VectorSubcoreMesh(core_axis_name="c", subcore_axis_name="s")
    fn = pl.kernel(
        _sc_body,
        out_type=jax.ShapeDtypeStruct((B, CP), jnp.float32),
        mesh=mesh,
        scratch_types=[
            pltpu.VMEM((IDXC + 16,), jnp.int32),
            pltpu.VMEM((IDXC,), jnp.int32),
            pltpu.VMEM((IDXC, 2 * D), jnp.float32),
            pltpu.VMEM((CHUNK, D), jnp.float32),
            pltpu.VMEM((CP * SKEW,), jnp.float32),
            pltpu.VMEM((CP * SKEW,), jnp.float32),
            pltpu.VMEM((CHUNK, CP), jnp.float32),
            pltpu.SemaphoreType.DMA,
        ],
        compiler_params=pltpu.CompilerParams(
            needs_layout_passes=False, use_tc_tiling_on_sc=True),
    )
    return fn(table2, widx_flat, ctx)


def _tc_body(ctx_ref, w_ref, b_ref, sco_ref, scores_ref, probs_ref, mt_ref):
    z = jnp.dot(ctx_ref[...], w_ref[...], preferred_element_type=jnp.float32)
    z = z + b_ref[...]
    mt_ref[...] = jax.nn.sigmoid(z)
    s = sco_ref[...][:, :C]
    m = jnp.max(s, axis=1, keepdims=True)
    e = jnp.exp(s - m)
    probs_ref[...] = e / jnp.sum(e, axis=1, keepdims=True)
    scores_ref[...] = s


def _tc_finish(ctx, type_W, type_b, sco_pad):
    nblk = 8
    blk = B // nblk
    return pl.pallas_call(
        _tc_body,
        grid=(nblk,),
        in_specs=[
            pl.BlockSpec((blk, D), lambda i: (i, 0)),
            pl.BlockSpec((D, NT), lambda i: (0, 0)),
            pl.BlockSpec((1, NT), lambda i: (0, 0)),
            pl.BlockSpec((blk, CP), lambda i: (i, 0)),
        ],
        out_specs=[
            pl.BlockSpec((blk, C), lambda i: (i, 0)),
            pl.BlockSpec((blk, C), lambda i: (i, 0)),
            pl.BlockSpec((blk, NT), lambda i: (i, 0)),
        ],
        out_shape=[
            jax.ShapeDtypeStruct((B, C), jnp.float32),
            jax.ShapeDtypeStruct((B, C), jnp.float32),
            jax.ShapeDtypeStruct((B, NT), jnp.float32),
        ],
    )(ctx, type_W, type_b.reshape(1, NT), sco_pad)


def kernel(leftb, rightb, leftlens, rightlens, docb, wididxsb,
           entity_table, context_encoded, type_W, type_b):
    widx_flat = wididxsb.reshape(-1)
    table2 = entity_table.reshape(ENVOC2, 2 * D)
    sco_pad = _sc_scores(table2, widx_flat, context_encoded)
    scores, probs, mtype = _tc_finish(context_encoded, type_W, type_b, sco_pad)
    return scores, probs, mtype


# speedup vs baseline: 1.8819x; 1.8819x over previous
import jax
import jax.numpy as jnp
from jax import lax
from jax.experimental import pallas as pl
from jax.experimental.pallas import tpu as pltpu
from jax.experimental.pallas import tpu_sc as plsc

ENVOC = 1000000
HALF = 503808
B = 4096
C = 30
CP = 32
D = 64
NT = 113

NC = 2
NS = 16
NW = NC * NS
BPW = B // NW
CHUNK = 16
NCHUNK = BPW // CHUNK
IDXC = CHUNK * C
GSL = 120
NG = IDXC // GSL
SKEW = 17

PBR = 4096
PGRID = HALF // PBR
MAXB = (ENVOC - 1) // PBR


def _pack_body(lo_ref, hi_ref, out_ref):
    out_ref[:, :D] = lo_ref[...].T
    out_ref[:, D:] = hi_ref[...].T


def _tc_pack(table_t):
    return pl.pallas_call(
        _pack_body,
        grid=(PGRID,),
        in_specs=[
            pl.BlockSpec((D, PBR), lambda i: (0, i)),
            pl.BlockSpec((D, PBR),
                         lambda i: (0, jnp.minimum(i + PGRID, MAXB))),
        ],
        out_specs=pl.BlockSpec((PBR, 2 * D), lambda i: (i, 0)),
        out_shape=jax.ShapeDtypeStruct((HALF, 2 * D), jnp.float32),
    )(table_t, table_t)


def _sc_body(table_hbm, widx_hbm, ctx_hbm, out_hbm,
             idx_v, pidx_v, rows_v, ctx_v, psa_v, psb_v, sco_v, sem):
    wid = lax.axis_index("s") * NC + lax.axis_index("c")
    zero16 = jnp.zeros((16,), jnp.float32)
    for pscr in (psa_v, psb_v):
        pscr[pl.ds(C * SKEW, 16)] = zero16
        pscr[pl.ds((C + 1) * SKEW, 16)] = zero16
    iota16 = lax.iota(jnp.int32, 16)
    tr_base = iota16 * SKEW

    for k in range(NCHUNK):
        bbase = wid * BPW + k * CHUNK
        ioff = pl.multiple_of(bbase * C, 8)
        pltpu.sync_copy(widx_hbm.at[pl.ds(ioff, IDXC)],
                        idx_v.at[pl.ds(0, IDXC)])
        pltpu.sync_copy(ctx_hbm.at[pl.ds(bbase, CHUNK)], ctx_v)
        for t in range(IDXC // 16):
            v = idx_v[pl.ds(t * 16, 16)]
            pidx_v[pl.ds(t * 16, 16)] = jnp.where(v >= HALF, v - HALF, v)
        cps = [
            pltpu.async_copy(
                table_hbm.at[pidx_v.at[pl.ds(j * GSL, GSL)]],
                rows_v.at[pl.ds(j * GSL, GSL)], sem)
            for j in range(NG)
        ]
        for cp in cps:
            cp.wait()

        def body_b(b, _):
            ctx = [ctx_v[b, pl.ds(i * 16, 16)] for i in range(4)]
            r0 = b * C
            for c in range(C):
                a0 = rows_v[r0 + c, pl.ds(0, 16)] * ctx[0]
                a1 = rows_v[r0 + c, pl.ds(D, 16)] * ctx[0]
                for i in range(1, 4):
                    a0 = a0 + rows_v[r0 + c, pl.ds(i * 16, 16)] * ctx[i]
                    a1 = a1 + rows_v[r0 + c, pl.ds(D + i * 16, 16)] * ctx[i]
                psa_v[pl.ds(c * SKEW, 16)] = a0
                psb_v[pl.ds(c * SKEW, 16)] = a1
            for g in range(2):
                s0 = jnp.zeros((16,), jnp.float32)
                s1 = jnp.zeros((16,), jnp.float32)
                for l in range(16):
                    tr = tr_base + (g * 16 * SKEW + l)
                    s0 = s0 + plsc.load_gather(psa_v, [tr])
                    s1 = s1 + plsc.load_gather(psb_v, [tr])
                hi = idx_v[pl.ds(r0 + g * 16, 16)] >= HALF
                sco_v[b, pl.ds(g * 16, 16)] = jnp.where(hi, s1, s0)
            return 0

        lax.fori_loop(0, CHUNK, body_b, 0)
        pltpu.sync_copy(sco_v, out_hbm.at[pl.ds(bbase, CHUNK)])


def _sc_scores(table2, widx_flat, ctx):
    mesh = plsc.VectorSubcoreMesh(core_axis_name="c", subcore_axis_name="s")
    fn = pl.kernel(
        _sc_body,
        out_type=jax.ShapeDtypeStruct((B, CP), jnp.float32),
        mesh=mesh,
        scratch_types=[
            pltpu.VMEM((IDXC + 16,), jnp.int32),
            pltpu.VMEM((IDXC,), jnp.int32),
            pltpu.VMEM((IDXC, 2 * D), jnp.float32),
            pltpu.VMEM((CHUNK, D), jnp.float32),
            pltpu.VMEM((CP * SKEW,), jnp.float32),
            pltpu.VMEM((CP * SKEW,), jnp.float32),
            pltpu.VMEM((CHUNK, CP), jnp.float32),
            pltpu.SemaphoreType.DMA,
        ],
        compiler_params=pltpu.CompilerParams(
            needs_layout_passes=False, use_tc_tiling_on_sc=True),
    )
    return fn(table2, widx_flat, ctx)


def _tc_body(ctx_ref, w_ref, b_ref, sco_ref, scores_ref, probs_ref, mt_ref):
    z = jnp.dot(ctx_ref[...], w_ref[...], preferred_element_type=jnp.float32)
    z = z + b_ref[...]
    mt_ref[...] = jax.nn.sigmoid(z)
    s = sco_ref[...][:, :C]
    m = jnp.max(s, axis=1, keepdims=True)
    e = jnp.exp(s - m)
    probs_ref[...] = e / jnp.sum(e, axis=1, keepdims=True)
    scores_ref[...] = s


def _tc_finish(ctx, type_W, type_b, sco_pad):
    nblk = 8
    blk = B // nblk
    return pl.pallas_call(
        _tc_body,
        grid=(nblk,),
        in_specs=[
            pl.BlockSpec((blk, D), lambda i: (i, 0)),
            pl.BlockSpec((D, NT), lambda i: (0, 0)),
            pl.BlockSpec((1, NT), lambda i: (0, 0)),
            pl.BlockSpec((blk, CP), lambda i: (i, 0)),
        ],
        out_specs=[
            pl.BlockSpec((blk, C), lambda i: (i, 0)),
            pl.BlockSpec((blk, C), lambda i: (i, 0)),
            pl.BlockSpec((blk, NT), lambda i: (i, 0)),
        ],
        out_shape=[
            jax.ShapeDtypeStruct((B, C), jnp.float32),
            jax.ShapeDtypeStruct((B, C), jnp.float32),
            jax.ShapeDtypeStruct((B, NT), jnp.float32),
        ],
    )(ctx, type_W, type_b.reshape(1, NT), sco_pad)


def kernel(leftb, rightb, leftlens, rightlens, docb, wididxsb,
           entity_table, context_encoded, type_W, type_b):
    widx_flat = wididxsb.reshape(-1)
    table2 = _tc_pack(entity_table.T)
    sco_pad = _sc_scores(table2, widx_flat, context_encoded)
    scores, probs, mtype = _tc_finish(context_encoded, type_W, type_b, sco_pad)
    return scores, probs, mtype


# speedup vs baseline: 1.9850x; 1.0548x over previous
import jax
import jax.numpy as jnp
from jax import lax
from jax.experimental import pallas as pl
from jax.experimental.pallas import tpu as pltpu
from jax.experimental.pallas import tpu_sc as plsc

ENVOC = 1000000
HALF = 503808
B = 4096
C = 30
CP = 32
D = 64
NT = 113

NC = 2
NS = 16
NW = NC * NS
BPW = B // NW
CHUNK = 8
NCHUNK = BPW // CHUNK
IDXC = CHUNK * C
GSL = 120
NG = IDXC // GSL
SKEW = 17

PBR = 4096
PGRID = HALF // PBR
MAXB = (ENVOC - 1) // PBR


def _pack_body(lo_ref, hi_ref, out_ref):
    out_ref[:, :D] = lo_ref[...].T
    out_ref[:, D:] = hi_ref[...].T


def _tc_pack(table_t):
    return pl.pallas_call(
        _pack_body,
        grid=(PGRID,),
        in_specs=[
            pl.BlockSpec((D, PBR), lambda i: (0, i)),
            pl.BlockSpec((D, PBR),
                         lambda i: (0, jnp.minimum(i + PGRID, MAXB))),
        ],
        out_specs=pl.BlockSpec((PBR, 2 * D), lambda i: (i, 0)),
        out_shape=jax.ShapeDtypeStruct((HALF, 2 * D), jnp.float32),
    )(table_t, table_t)


def _sc_body(table_hbm, widx_hbm, ctx_hbm, out_hbm,
             idx_v0, idx_v1, pidx_v0, pidx_v1, rows_v0, rows_v1,
             ctx_v0, ctx_v1, psa_v, psb_v, sco_v, sem0, sem1):
    wid = lax.axis_index("s") * NC + lax.axis_index("c")
    zero16 = jnp.zeros((16,), jnp.float32)
    for pscr in (psa_v, psb_v):
        pscr[pl.ds(C * SKEW, 16)] = zero16
        pscr[pl.ds((C + 1) * SKEW, 16)] = zero16
    iota16 = lax.iota(jnp.int32, 16)
    tr_base = iota16 * SKEW
    bufs = ((idx_v0, pidx_v0, rows_v0, ctx_v0, sem0),
            (idx_v1, pidx_v1, rows_v1, ctx_v1, sem1))

    def stage(kk, buf):
        idx_v, pidx_v, rows_v, ctx_v, sem = buf
        bbase = wid * BPW + kk * CHUNK
        ioff = pl.multiple_of(bbase * C, 8)
        pltpu.sync_copy(widx_hbm.at[pl.ds(ioff, IDXC)],
                        idx_v.at[pl.ds(0, IDXC)])
        pltpu.sync_copy(ctx_hbm.at[pl.ds(bbase, CHUNK)], ctx_v)
        for t in range(IDXC // 16):
            v = idx_v[pl.ds(t * 16, 16)]
            pidx_v[pl.ds(t * 16, 16)] = jnp.where(v >= HALF, v - HALF, v)
        for j in range(NG):
            pltpu.async_copy(
                table_hbm.at[pidx_v.at[pl.ds(j * GSL, GSL)]],
                rows_v.at[pl.ds(j * GSL, GSL)], sem)

    def drain(buf):
        rows_v, sem = buf[2], buf[4]
        pltpu.make_async_copy(
            table_hbm.at[pl.ds(0, IDXC)], rows_v, sem).wait()

    def compute(kk, buf):
        idx_v, _, rows_v, ctx_v, _ = buf
        bbase = wid * BPW + kk * CHUNK

        def body_b(b, _):
            ctx = [ctx_v[b, pl.ds(i * 16, 16)] for i in range(4)]
            r0 = b * C
            for c in range(C):
                a0 = rows_v[r0 + c, pl.ds(0, 16)] * ctx[0]
                a1 = rows_v[r0 + c, pl.ds(D, 16)] * ctx[0]
                for i in range(1, 4):
                    a0 = a0 + rows_v[r0 + c, pl.ds(i * 16, 16)] * ctx[i]
                    a1 = a1 + rows_v[r0 + c, pl.ds(D + i * 16, 16)] * ctx[i]
                psa_v[pl.ds(c * SKEW, 16)] = a0
                psb_v[pl.ds(c * SKEW, 16)] = a1
            for g in range(2):
                s0 = jnp.zeros((16,), jnp.float32)
                s1 = jnp.zeros((16,), jnp.float32)
                for l in range(16):
                    tr = tr_base + (g * 16 * SKEW + l)
                    s0 = s0 + plsc.load_gather(psa_v, [tr])
                    s1 = s1 + plsc.load_gather(psb_v, [tr])
                hi = idx_v[pl.ds(r0 + g * 16, 16)] >= HALF
                sco_v[b, pl.ds(g * 16, 16)] = jnp.where(hi, s1, s0)
            return 0

        lax.fori_loop(0, CHUNK, body_b, 0)
        pltpu.sync_copy(sco_v, out_hbm.at[pl.ds(bbase, CHUNK)])

    stage(0, bufs[0])

    def body_j(j, _):
        for half in range(2):
            kk = 2 * j + half
            drain(bufs[half])
            nxt = kk + 1

            @pl.when(nxt < NCHUNK)
            def _():
                stage(nxt, bufs[1 - half])

            compute(kk, bufs[half])
        return 0

    lax.fori_loop(0, NCHUNK // 2, body_j, 0)


def _sc_scores(table2, widx_flat, ctx):
    mesh = plsc.VectorSubcoreMesh(core_axis_name="c", subcore_axis_name="s")
    fn = pl.kernel(
        _sc_body,
        out_type=jax.ShapeDtypeStruct((B, CP), jnp.float32),
        mesh=mesh,
        scratch_types=[
            pltpu.VMEM((IDXC + 16,), jnp.int32),
            pltpu.VMEM((IDXC + 16,), jnp.int32),
            pltpu.VMEM((IDXC,), jnp.int32),
            pltpu.VMEM((IDXC,), jnp.int32),
            pltpu.VMEM((IDXC, 2 * D), jnp.float32),
            pltpu.VMEM((IDXC, 2 * D), jnp.float32),
            pltpu.VMEM((CHUNK, D), jnp.float32),
            pltpu.VMEM((CHUNK, D), jnp.float32),
            pltpu.VMEM((CP * SKEW,), jnp.float32),
            pltpu.VMEM((CP * SKEW,), jnp.float32),
            pltpu.VMEM((CHUNK, CP), jnp.float32),
            pltpu.SemaphoreType.DMA,
            pltpu.SemaphoreType.DMA,
        ],
        compiler_params=pltpu.CompilerParams(
            needs_layout_passes=False, use_tc_tiling_on_sc=True),
    )
    return fn(table2, widx_flat, ctx)


def _tc_body(ctx_ref, w_ref, b_ref, sco_ref, scores_ref, probs_ref, mt_ref):
    z = jnp.dot(ctx_ref[...], w_ref[...], preferred_element_type=jnp.float32)
    z = z + b_ref[...]
    mt_ref[...] = jax.nn.sigmoid(z)
    s = sco_ref[...][:, :C]
    m = jnp.max(s, axis=1, keepdims=True)
    e = jnp.exp(s - m)
    probs_ref[...] = e / jnp.sum(e, axis=1, keepdims=True)
    scores_ref[...] = s


def _tc_finish(ctx, type_W, type_b, sco_pad):
    nblk = 8
    blk = B // nblk
    return pl.pallas_call(
        _tc_body,
        grid=(nblk,),
        in_specs=[
            pl.BlockSpec((blk, D), lambda i: (i, 0)),
            pl.BlockSpec((D, NT), lambda i: (0, 0)),
            pl.BlockSpec((1, NT), lambda i: (0, 0)),
            pl.BlockSpec((blk, CP), lambda i: (i, 0)),
        ],
        out_specs=[
            pl.BlockSpec((blk, C), lambda i: (i, 0)),
            pl.BlockSpec((blk, C), lambda i: (i, 0)),
            pl.BlockSpec((blk, NT), lambda i: (i, 0)),
        ],
        out_shape=[
            jax.ShapeDtypeStruct((B, C), jnp.float32),
            jax.ShapeDtypeStruct((B, C), jnp.float32),
            jax.ShapeDtypeStruct((B, NT), jnp.float32),
        ],
    )(ctx, type_W, type_b.reshape(1, NT), sco_pad)


def kernel(leftb, rightb, leftlens, rightlens, docb, wididxsb,
           entity_table, context_encoded, type_W, type_b):
    widx_flat = wididxsb.reshape(-1)
    table2 = _tc_pack(entity_table.T)
    sco_pad = _sc_scores(table2, widx_flat, context_encoded)
    scores, probs, mtype = _tc_finish(context_encoded, type_W, type_b, sco_pad)
    return scores, probs, mtype


# speedup vs baseline: 2.1787x; 1.0976x over previous
import jax
import jax.numpy as jnp
from jax import lax
from jax.experimental import pallas as pl
from jax.experimental.pallas import tpu as pltpu
from jax.experimental.pallas import tpu_sc as plsc

ENVOC = 1000000
HALF = 507904
B = 4096
C = 30
CP = 32
D = 64
NT = 113

NC = 2
NS = 16
NW = NC * NS
BPW = B // NW
CHUNK = 8
NCHUNK = BPW // CHUNK
IDXC = CHUNK * C
GSL = 120
NG = IDXC // GSL
SKEW = 17

PBR = 8192
PGRID = HALF // PBR
MAXB = (ENVOC - 1) // PBR


def _pack_body(lo_ref, hi_ref, out_ref):
    out_ref[:, :D] = lo_ref[...].T
    out_ref[:, D:] = hi_ref[...].T


def _tc_pack(table_t):
    return pl.pallas_call(
        _pack_body,
        grid=(PGRID,),
        in_specs=[
            pl.BlockSpec((D, PBR), lambda i: (0, i)),
            pl.BlockSpec((D, PBR),
                         lambda i: (0, jnp.minimum(i + PGRID, MAXB))),
        ],
        out_specs=pl.BlockSpec((PBR, 2 * D), lambda i: (i, 0)),
        out_shape=jax.ShapeDtypeStruct((HALF, 2 * D), jnp.float32),
    )(table_t, table_t)


def _sc_body(table_hbm, widx_hbm, ctx_hbm, out_hbm,
             idx_v0, idx_v1, pidx_v0, pidx_v1, rows_v0, rows_v1,
             ctx_v0, ctx_v1, psa_v, psb_v, sco_v, sem0, sem1):
    wid = lax.axis_index("s") * NC + lax.axis_index("c")
    zero16 = jnp.zeros((16,), jnp.float32)
    for pscr in (psa_v, psb_v):
        pscr[pl.ds(C * SKEW, 16)] = zero16
        pscr[pl.ds((C + 1) * SKEW, 16)] = zero16
    iota16 = lax.iota(jnp.int32, 16)
    tr_base = iota16 * SKEW
    bufs = ((idx_v0, pidx_v0, rows_v0, ctx_v0, sem0),
            (idx_v1, pidx_v1, rows_v1, ctx_v1, sem1))

    def stage(kk, buf):
        idx_v, pidx_v, rows_v, ctx_v, sem = buf
        bbase = wid * BPW + kk * CHUNK
        ioff = pl.multiple_of(bbase * C, 8)
        pltpu.sync_copy(widx_hbm.at[pl.ds(ioff, IDXC)],
                        idx_v.at[pl.ds(0, IDXC)])
        pltpu.sync_copy(ctx_hbm.at[pl.ds(bbase, CHUNK)], ctx_v)
        for t in range(IDXC // 16):
            v = idx_v[pl.ds(t * 16, 16)]
            pidx_v[pl.ds(t * 16, 16)] = jnp.where(v >= HALF, v - HALF, v)
        for j in range(NG):
            pltpu.async_copy(
                table_hbm.at[pidx_v.at[pl.ds(j * GSL, GSL)]],
                rows_v.at[pl.ds(j * GSL, GSL)], sem)

    def drain(buf):
        rows_v, sem = buf[2], buf[4]
        pltpu.make_async_copy(
            table_hbm.at[pl.ds(0, IDXC)], rows_v, sem).wait()

    def compute(kk, buf):
        idx_v, _, rows_v, ctx_v, _ = buf
        bbase = wid * BPW + kk * CHUNK

        def body_b(b, _):
            ctx = [ctx_v[b, pl.ds(i * 16, 16)] for i in range(4)]
            r0 = b * C
            for c in range(C):
                a0 = rows_v[r0 + c, pl.ds(0, 16)] * ctx[0]
                a1 = rows_v[r0 + c, pl.ds(D, 16)] * ctx[0]
                for i in range(1, 4):
                    a0 = a0 + rows_v[r0 + c, pl.ds(i * 16, 16)] * ctx[i]
                    a1 = a1 + rows_v[r0 + c, pl.ds(D + i * 16, 16)] * ctx[i]
                psa_v[pl.ds(c * SKEW, 16)] = a0
                psb_v[pl.ds(c * SKEW, 16)] = a1
            for g in range(2):
                s0 = jnp.zeros((16,), jnp.float32)
                s1 = jnp.zeros((16,), jnp.float32)
                for l in range(16):
                    tr = tr_base + (g * 16 * SKEW + l)
                    s0 = s0 + plsc.load_gather(psa_v, [tr])
                    s1 = s1 + plsc.load_gather(psb_v, [tr])
                hi = idx_v[pl.ds(r0 + g * 16, 16)] >= HALF
                sco_v[b, pl.ds(g * 16, 16)] = jnp.where(hi, s1, s0)
            return 0

        lax.fori_loop(0, CHUNK, body_b, 0)
        pltpu.sync_copy(sco_v, out_hbm.at[pl.ds(bbase, CHUNK)])

    stage(0, bufs[0])

    def body_j(j, _):
        for half in range(2):
            kk = 2 * j + half
            drain(bufs[half])
            nxt = kk + 1

            @pl.when(nxt < NCHUNK)
            def _():
                stage(nxt, bufs[1 - half])

            compute(kk, bufs[half])
        return 0

    lax.fori_loop(0, NCHUNK // 2, body_j, 0)


def _sc_scores(table2, widx_flat, ctx):
    mesh = plsc.VectorSubcoreMesh(core_axis_name="c", subcore_axis_name="s")
    fn = pl.kernel(
        _sc_body,
        out_type=jax.ShapeDtypeStruct((B, CP), jnp.float32),
        mesh=mesh,
        scratch_types=[
            pltpu.VMEM((IDXC + 16,), jnp.int32),
            pltpu.VMEM((IDXC + 16,), jnp.int32),
            pltpu.VMEM((IDXC,), jnp.int32),
            pltpu.VMEM((IDXC,), jnp.int32),
            pltpu.VMEM((IDXC, 2 * D), jnp.float32),
            pltpu.VMEM((IDXC, 2 * D), jnp.float32),
            pltpu.VMEM((CHUNK, D), jnp.float32),
            pltpu.VMEM((CHUNK, D), jnp.float32),
            pltpu.VMEM((CP * SKEW,), jnp.float32),
            pltpu.VMEM((CP * SKEW,), jnp.float32),
            pltpu.VMEM((CHUNK, CP), jnp.float32),
            pltpu.SemaphoreType.DMA,
            pltpu.SemaphoreType.DMA,
        ],
        compiler_params=pltpu.CompilerParams(
            needs_layout_passes=False, use_tc_tiling_on_sc=True),
    )
    return fn(table2, widx_flat, ctx)


def _tc_body(ctx_ref, w_ref, b_ref, sco_ref, scores_ref, probs_ref, mt_ref):
    z = jnp.dot(ctx_ref[...], w_ref[...], preferred_element_type=jnp.float32)
    z = z + b_ref[...]
    mt_ref[...] = jax.nn.sigmoid(z)
    s = sco_ref[...][:, :C]
    m = jnp.max(s, axis=1, keepdims=True)
    e = jnp.exp(s - m)
    probs_ref[...] = e / jnp.sum(e, axis=1, keepdims=True)
    scores_ref[...] = s


def _tc_finish(ctx, type_W, type_b, sco_pad):
    nblk = 8
    blk = B // nblk
    return pl.pallas_call(
        _tc_body,
        grid=(nblk,),
        in_specs=[
            pl.BlockSpec((blk, D), lambda i: (i, 0)),
            pl.BlockSpec((D, NT), lambda i: (0, 0)),
            pl.BlockSpec((1, NT), lambda i: (0, 0)),
            pl.BlockSpec((blk, CP), lambda i: (i, 0)),
        ],
        out_specs=[
            pl.BlockSpec((blk, C), lambda i: (i, 0)),
            pl.BlockSpec((blk, C), lambda i: (i, 0)),
            pl.BlockSpec((blk, NT), lambda i: (i, 0)),
        ],
        out_shape=[
            jax.ShapeDtypeStruct((B, C), jnp.float32),
            jax.ShapeDtypeStruct((B, C), jnp.float32),
            jax.ShapeDtypeStruct((B, NT), jnp.float32),
        ],
    )(ctx, type_W, type_b.reshape(1, NT), sco_pad)


def kernel(leftb, rightb, leftlens, rightlens, docb, wididxsb,
           entity_table, context_encoded, type_W, type_b):
    widx_flat = wididxsb.reshape(-1)
    table2 = _tc_pack(entity_table.T)
    sco_pad = _sc_scores(table2, widx_flat, context_encoded)
    scores, probs, mtype = _tc_finish(context_encoded, type_W, type_b, sco_pad)
    return scores, probs, mtype


# speedup vs baseline: 2.2761x; 1.0447x over previous
import jax
import jax.numpy as jnp
from jax import lax
from jax.experimental import pallas as pl
from jax.experimental.pallas import tpu as pltpu
from jax.experimental.pallas import tpu_sc as plsc

ENVOC = 1000000
HALF = 507904
B = 4096
C = 30
CP = 32
D = 64
NT = 113

NC = 2
NS = 16
NW = NC * NS
BPW = B // NW
CHUNK = 8
NCHUNK = BPW // CHUNK
IDXC = CHUNK * C
GSL = 120
NG = IDXC // GSL
SKEW = 17

PBR = 16384
PGRID = HALF // PBR
MAXB = (ENVOC - 1) // PBR


def _pack_body(lo_ref, hi_ref, out_ref):
    out_ref[:, :D] = lo_ref[...].T
    out_ref[:, D:] = hi_ref[...].T


def _tc_pack(table_t):
    return pl.pallas_call(
        _pack_body,
        grid=(PGRID,),
        in_specs=[
            pl.BlockSpec((D, PBR), lambda i: (0, i)),
            pl.BlockSpec((D, PBR),
                         lambda i: (0, jnp.minimum(i + PGRID, MAXB))),
        ],
        out_specs=pl.BlockSpec((PBR, 2 * D), lambda i: (i, 0)),
        out_shape=jax.ShapeDtypeStruct((HALF, 2 * D), jnp.float32),
    )(table_t, table_t)


def _sc_body(table_hbm, widx_hbm, ctx_hbm, out_hbm,
             idx_v0, idx_v1, pidx_v0, pidx_v1, rows_v0, rows_v1,
             ctx_v0, ctx_v1, psa_v, psb_v, sco_v, sem0, sem1):
    wid = lax.axis_index("s") * NC + lax.axis_index("c")
    zero16 = jnp.zeros((16,), jnp.float32)
    for pscr in (psa_v, psb_v):
        pscr[pl.ds(C * SKEW, 16)] = zero16
        pscr[pl.ds((C + 1) * SKEW, 16)] = zero16
    iota16 = lax.iota(jnp.int32, 16)
    tr_base = iota16 * SKEW
    bufs = ((idx_v0, pidx_v0, rows_v0, ctx_v0, sem0),
            (idx_v1, pidx_v1, rows_v1, ctx_v1, sem1))

    def stage(kk, buf):
        idx_v, pidx_v, rows_v, ctx_v, sem = buf
        bbase = wid * BPW + kk * CHUNK
        ioff = pl.multiple_of(bbase * C, 8)
        pltpu.sync_copy(widx_hbm.at[pl.ds(ioff, IDXC)],
                        idx_v.at[pl.ds(0, IDXC)])
        pltpu.sync_copy(ctx_hbm.at[pl.ds(bbase, CHUNK)], ctx_v)
        for t in range(IDXC // 16):
            v = idx_v[pl.ds(t * 16, 16)]
            pidx_v[pl.ds(t * 16, 16)] = jnp.where(v >= HALF, v - HALF, v)
        for j in range(NG):
            pltpu.async_copy(
                table_hbm.at[pidx_v.at[pl.ds(j * GSL, GSL)]],
                rows_v.at[pl.ds(j * GSL, GSL)], sem)

    def drain(buf):
        rows_v, sem = buf[2], buf[4]
        pltpu.make_async_copy(
            table_hbm.at[pl.ds(0, IDXC)], rows_v, sem).wait()

    def compute(kk, buf):
        idx_v, _, rows_v, ctx_v, _ = buf
        bbase = wid * BPW + kk * CHUNK

        def body_b(b, _):
            ctx = [ctx_v[b, pl.ds(i * 16, 16)] for i in range(4)]
            r0 = b * C
            for c in range(C):
                a0 = rows_v[r0 + c, pl.ds(0, 16)] * ctx[0]
                a1 = rows_v[r0 + c, pl.ds(D, 16)] * ctx[0]
                for i in range(1, 4):
                    a0 = a0 + rows_v[r0 + c, pl.ds(i * 16, 16)] * ctx[i]
                    a1 = a1 + rows_v[r0 + c, pl.ds(D + i * 16, 16)] * ctx[i]
                psa_v[pl.ds(c * SKEW, 16)] = a0
                psb_v[pl.ds(c * SKEW, 16)] = a1
            for g in range(2):
                s0 = jnp.zeros((16,), jnp.float32)
                s1 = jnp.zeros((16,), jnp.float32)
                for l in range(16):
                    tr = tr_base + (g * 16 * SKEW + l)
                    s0 = s0 + plsc.load_gather(psa_v, [tr])
                    s1 = s1 + plsc.load_gather(psb_v, [tr])
                hi = idx_v[pl.ds(r0 + g * 16, 16)] >= HALF
                sco_v[b, pl.ds(g * 16, 16)] = jnp.where(hi, s1, s0)
            return 0

        lax.fori_loop(0, CHUNK, body_b, 0)
        pltpu.sync_copy(sco_v, out_hbm.at[pl.ds(bbase, CHUNK)])

    stage(0, bufs[0])

    def body_j(j, _):
        for half in range(2):
            kk = 2 * j + half
            drain(bufs[half])
            nxt = kk + 1

            @pl.when(nxt < NCHUNK)
            def _():
                stage(nxt, bufs[1 - half])

            compute(kk, bufs[half])
        return 0

    lax.fori_loop(0, NCHUNK // 2, body_j, 0)


def _sc_scores(table2, widx_flat, ctx):
    mesh = plsc.VectorSubcoreMesh(core_axis_name="c", subcore_axis_name="s")
    fn = pl.kernel(
        _sc_body,
        out_type=jax.ShapeDtypeStruct((B, CP), jnp.float32),
        mesh=mesh,
        scratch_types=[
            pltpu.VMEM((IDXC + 16,), jnp.int32),
            pltpu.VMEM((IDXC + 16,), jnp.int32),
            pltpu.VMEM((IDXC,), jnp.int32),
            pltpu.VMEM((IDXC,), jnp.int32),
            pltpu.VMEM((IDXC, 2 * D), jnp.float32),
            pltpu.VMEM((IDXC, 2 * D), jnp.float32),
            pltpu.VMEM((CHUNK, D), jnp.float32),
            pltpu.VMEM((CHUNK, D), jnp.float32),
            pltpu.VMEM((CP * SKEW,), jnp.float32),
            pltpu.VMEM((CP * SKEW,), jnp.float32),
            pltpu.VMEM((CHUNK, CP), jnp.float32),
            pltpu.SemaphoreType.DMA,
            pltpu.SemaphoreType.DMA,
        ],
        compiler_params=pltpu.CompilerParams(
            needs_layout_passes=False, use_tc_tiling_on_sc=True),
    )
    return fn(table2, widx_flat, ctx)


def _tc_body(ctx_ref, w_ref, b_ref, sco_ref, scores_ref, probs_ref, mt_ref):
    z = jnp.dot(ctx_ref[...], w_ref[...], preferred_element_type=jnp.float32)
    z = z + b_ref[...]
    mt_ref[...] = jax.nn.sigmoid(z)
    s = sco_ref[...][:, :C]
    m = jnp.max(s, axis=1, keepdims=True)
    e = jnp.exp(s - m)
    probs_ref[...] = e / jnp.sum(e, axis=1, keepdims=True)
    scores_ref[...] = s


def _tc_finish(ctx, type_W, type_b, sco_pad):
    nblk = 8
    blk = B // nblk
    return pl.pallas_call(
        _tc_body,
        grid=(nblk,),
        in_specs=[
            pl.BlockSpec((blk, D), lambda i: (i, 0)),
            pl.BlockSpec((D, NT), lambda i: (0, 0)),
            pl.BlockSpec((1, NT), lambda i: (0, 0)),
            pl.BlockSpec((blk, CP), lambda i: (i, 0)),
        ],
        out_specs=[
            pl.BlockSpec((blk, C), lambda i: (i, 0)),
            pl.BlockSpec((blk, C), lambda i: (i, 0)),
            pl.BlockSpec((blk, NT), lambda i: (i, 0)),
        ],
        out_shape=[
            jax.ShapeDtypeStruct((B, C), jnp.float32),
            jax.ShapeDtypeStruct((B, C), jnp.float32),
            jax.ShapeDtypeStruct((B, NT), jnp.float32),
        ],
    )(ctx, type_W, type_b.reshape(1, NT), sco_pad)


def kernel(leftb, rightb, leftlens, rightlens, docb, wididxsb,
           entity_table, context_encoded, type_W, type_b):
    widx_flat = wididxsb.reshape(-1)
    table2 = _tc_pack(entity_table.T)
    sco_pad = _sc_scores(table2, widx_flat, context_encoded)
    scores, probs, mtype = _tc_finish(context_encoded, type_W, type_b, sco_pad)
    return scores, probs, mtype
